# baseline (speedup 1.0000x reference)
import jax
import jax.numpy as jnp
from jax import lax
from jax.experimental import pallas as pl
from jax.experimental.pallas import tpu as pltpu
from jax.experimental.pallas import tpu_sc as plsc

NC = 2
NS = 16
L = 16

H_SP = 64
W_SP = 64
HEADS = 8
D = 32
Q = 4096
P = 4
K = H_SP * W_SP
PAIRS = 32
NWORD = D // 2
STRIDE = NWORD + 1
R = 128
GROUPS = R // L
NCHUNK = Q // R
VSTG = 128


def _floor_i32(v):
    t = v.astype(jnp.int32)
    tf = t.astype(jnp.float32)
    return jnp.where(v < tf, t - 1, t)


def _body(vtbl_hbm, loc_hbm, aw_hbm, out_hbm,
          slab_v, vstage, idx_v, loc_c, aw_c, out_v, sem):
    wid = lax.axis_index("c") * NS + lax.axis_index("s")
    b = lax.shift_right_logical(wid, 3)
    h = lax.bitwise_and(wid, HEADS - 1)
    base = b * (K * 2) + lax.shift_right_logical(h, 2)
    hoff = lax.bitwise_and(h, 3) * D
    lanes = lax.iota(jnp.int32, L)
    lanes2 = lanes * 2

    def stage_body(cc, carry):
        for j in range(VSTG // L):
            idx_v[pl.ds(j * L, L)] = base + (cc * VSTG + j * L + lanes) * 2
        pltpu.async_copy(vtbl_hbm.at[idx_v], vstage, sem).wait()

        def pack_body(t, pcarry):
            tvec = jnp.zeros((L,), jnp.int32) + t
            ev = plsc.load_gather(vstage, [tvec, hoff + lanes2])
            od = plsc.load_gather(vstage, [tvec, hoff + lanes2 + 1])
            w = plsc.pack(ev, od, format=plsc.PackFormat.INTERLEAVED)
            slab_v[pl.ds((cc * VSTG + t) * STRIDE, NWORD)] = plsc.bitcast(
                w, jnp.int32)
            return pcarry

        lax.fori_loop(0, VSTG, pack_body, 0, unroll=8)
        return carry

    lax.fori_loop(0, K // VSTG, stage_body, 0)

    zz = jnp.zeros((L,), jnp.int32)
    oo = jnp.ones((L,), jnp.int32)

    h8 = h * 8
    h4 = h * 4

    def chunk_body(c, carry):
        pltpu.sync_copy(loc_hbm.at[b, pl.ds(c * (R * 64), R * 64)], loc_c)
        pltpu.sync_copy(aw_hbm.at[b, pl.ds(c * (R * 32), R * 32)], aw_c)

        def group_body(g, gcarry):
            rows = g * L + lanes
            rows64 = rows * 64 + h8
            rows32a = rows * 32 + h4
            wbfs = []
            tbases = []
            for p in range(P):
                xx = plsc.load_gather(loc_c, [rows64 + 2 * p])
                yy = plsc.load_gather(loc_c, [rows64 + 2 * p + 1])
                aa = plsc.load_gather(aw_c, [rows32a + p])
                ix = xx * jnp.float32(W_SP) - 0.5
                iy = yy * jnp.float32(H_SP) - 0.5
                x0 = _floor_i32(ix)
                y0 = _floor_i32(iy)
                wx1 = ix - x0.astype(jnp.float32)
                wx0 = 1.0 - wx1
                wy1 = iy - y0.astype(jnp.float32)
                wy0 = 1.0 - wy1
                for sy in (0, 1):
                    yc = y0 + sy
                    wy = wy1 if sy else wy0
                    vy = (yc >= 0) & (yc <= H_SP - 1)
                    ycc = jnp.clip(yc, 0, H_SP - 1)
                    for sx in (0, 1):
                        xc = x0 + sx
                        wx = wx1 if sx else wx0
                        ok = vy & (xc >= 0) & (xc <= W_SP - 1)
                        xcc = jnp.clip(xc, 0, W_SP - 1)
                        t = ycc * W_SP + xcc
                        w = jnp.where(ok, aa * wx * wy, 0.0)
                        wbfs.append(
                            plsc.pack(w, w, format=plsc.PackFormat.INTERLEAVED)
                        )
                        tbases.append(t * STRIDE)
            for wp in range(NWORD):
                acc = jnp.zeros((2 * L,), jnp.bfloat16)
                for s in range(16):
                    word = plsc.load_gather(slab_v, [tbases[s] + wp])
                    acc = acc + plsc.bitcast(word, jnp.bfloat16) * wbfs[s]
                even, odd = plsc.unpack(acc, format=plsc.PackFormat.INTERLEAVED)
                plsc.store_scatter(out_v, [rows * D + 2 * wp], even)
                plsc.store_scatter(out_v, [rows * D + 2 * wp + 1], odd)
            return gcarry

        lax.fori_loop(0, GROUPS, group_body, 0)
        pltpu.sync_copy(out_v, out_hbm.at[wid, pl.ds(c * (R * D), R * D)])
        return carry

    lax.fori_loop(0, NCHUNK, chunk_body, 0)


@jax.jit
def _run(vtbl, loc, aw):
    kfn = pl.kernel(
        _body,
        out_type=jax.ShapeDtypeStruct((PAIRS, Q * D), jnp.float32),
        mesh=plsc.VectorSubcoreMesh(
            core_axis_name="c", subcore_axis_name="s",
            num_cores=NC, num_subcores=NS,
        ),
        scratch_types=[
            pltpu.VMEM((K * STRIDE,), jnp.int32),
            pltpu.VMEM((VSTG, 4 * D), jnp.float32),
            pltpu.VMEM((VSTG,), jnp.int32),
            pltpu.VMEM((R * HEADS * P * 2,), jnp.float32),
            pltpu.VMEM((R * HEADS * P,), jnp.float32),
            pltpu.VMEM((R * D,), jnp.float32),
            pltpu.SemaphoreType.DMA,
        ],
        compiler_params=pltpu.CompilerParams(needs_layout_passes=False),
    )
    out = kfn(vtbl, loc, aw)
    return (
        out.reshape(4, HEADS, Q, D)
        .transpose(0, 2, 1, 3)
        .reshape(4, Q, HEADS * D)
    )


def kernel(value, value_spatial_shapes, sampling_locations, attention_weights):
    bs, _, heads, d = value.shape
    vtbl = value.reshape(bs * K * heads // 4, 4 * d)
    loc = sampling_locations.reshape(bs, Q * heads * P * 2)
    aw = attention_weights.reshape(bs, Q * heads * P)
    return _run(vtbl, loc, aw)


# speedup vs baseline: 2.1983x; 2.1983x over previous
import jax
import jax.numpy as jnp
from jax import lax
from jax.experimental import pallas as pl
from jax.experimental.pallas import tpu as pltpu
from jax.experimental.pallas import tpu_sc as plsc

NC = 2
NS = 16
L = 16

H_SP = 64
W_SP = 64
HEADS = 8
D = 32
Q = 4096
P = 4
K = H_SP * W_SP
PAIRS = 32
NWORD = D // 2
R = 128
GROUPS = R // L
NCHUNK = Q // R
PLSTEP = 8


def _floor_i32(v):
    t = v.astype(jnp.int32)
    tf = t.astype(jnp.float32)
    return jnp.where(v < tf, t - 1, t)


def _body(vt_hbm, lc_hbm, aw_hbm, out_hbm,
          slab_v, vstage, loc_c, aw_c, out_v):
    wid = lax.axis_index("c") * NS + lax.axis_index("s")
    b = lax.shift_right_logical(wid, 3)
    h = lax.bitwise_and(wid, HEADS - 1)
    lanes = lax.iota(jnp.int32, L)

    for qc in range(D // PLSTEP):
        pltpu.sync_copy(vt_hbm.at[b, h, pl.ds(qc * PLSTEP, PLSTEP)], vstage)
        for wl in range(PLSTEP // 2):
            wp = qc * (PLSTEP // 2) + wl

            def pack_body(k, pcarry, wl=wl, wp=wp):
                ev = vstage[2 * wl, pl.ds(k * L, L)]
                od = vstage[2 * wl + 1, pl.ds(k * L, L)]
                w = plsc.pack(ev, od, format=plsc.PackFormat.INTERLEAVED)
                slab_v[pl.ds(wp * Q + k * L, L)] = plsc.bitcast(w, jnp.int32)
                return pcarry

            lax.fori_loop(0, Q // L, pack_body, 0, unroll=8)

    def chunk_body(c, carry):
        pltpu.sync_copy(lc_hbm.at[b, h, 0, :, :, pl.ds(c * R, R)], loc_c)
        pltpu.sync_copy(aw_hbm.at[b, h, 0, :, pl.ds(c * R, R)], aw_c)

        def group_body(g, gcarry):
            rows = g * L + lanes
            wbfs = []
            tbases = []
            for p in range(P):
                xx = loc_c[p, 0, pl.ds(g * L, L)]
                yy = loc_c[p, 1, pl.ds(g * L, L)]
                aa = aw_c[p, pl.ds(g * L, L)]
                ix = xx * jnp.float32(W_SP) - 0.5
                iy = yy * jnp.float32(H_SP) - 0.5
                x0 = _floor_i32(ix)
                y0 = _floor_i32(iy)
                wx1 = ix - x0.astype(jnp.float32)
                wx0 = 1.0 - wx1
                wy1 = iy - y0.astype(jnp.float32)
                wy0 = 1.0 - wy1
                for sy in (0, 1):
                    yc = y0 + sy
                    wy = wy1 if sy else wy0
                    vy = (yc >= 0) & (yc <= H_SP - 1)
                    ycc = jnp.clip(yc, 0, H_SP - 1)
                    for sx in (0, 1):
                        xc = x0 + sx
                        wx = wx1 if sx else wx0
                        ok = vy & (xc >= 0) & (xc <= W_SP - 1)
                        xcc = jnp.clip(xc, 0, W_SP - 1)
                        t = ycc * W_SP + xcc
                        w = jnp.where(ok, aa * wx * wy, 0.0)
                        wbfs.append(
                            plsc.pack(w, w, format=plsc.PackFormat.INTERLEAVED)
                        )
                        tbases.append(t)
            for wp in range(NWORD):
                acc = jnp.zeros((2 * L,), jnp.bfloat16)
                for s in range(16):
                    word = plsc.load_gather(slab_v, [tbases[s] + wp * Q])
                    acc = acc + plsc.bitcast(word, jnp.bfloat16) * wbfs[s]
                even, odd = plsc.unpack(acc, format=plsc.PackFormat.INTERLEAVED)
                plsc.store_scatter(out_v, [rows * D + 2 * wp], even)
                plsc.store_scatter(out_v, [rows * D + 2 * wp + 1], odd)
            return gcarry

        lax.fori_loop(0, GROUPS, group_body, 0)
        pltpu.sync_copy(out_v, out_hbm.at[wid, pl.ds(c * (R * D), R * D)])
        return carry

    lax.fori_loop(0, NCHUNK, chunk_body, 0)


@jax.jit
def _run(value, loc6, aw5):
    vt = value.transpose(0, 2, 3, 1)
    lc = loc6.transpose(0, 2, 3, 4, 5, 1)
    aw = aw5.transpose(0, 2, 3, 4, 1)
    kfn = pl.kernel(
        _body,
        out_type=jax.ShapeDtypeStruct((PAIRS, Q * D), jnp.float32),
        mesh=plsc.VectorSubcoreMesh(
            core_axis_name="c", subcore_axis_name="s",
            num_cores=NC, num_subcores=NS,
        ),
        scratch_types=[
            pltpu.VMEM((NWORD * Q,), jnp.int32),
            pltpu.VMEM((PLSTEP, Q), jnp.float32),
            pltpu.VMEM((P, 2, R), jnp.float32),
            pltpu.VMEM((P, R), jnp.float32),
            pltpu.VMEM((R * D,), jnp.float32),
        ],
        compiler_params=pltpu.CompilerParams(needs_layout_passes=False),
    )
    out = kfn(vt, lc, aw)
    return (
        out.reshape(4, HEADS, Q, D)
        .transpose(0, 2, 1, 3)
        .reshape(4, Q, HEADS * D)
    )


def kernel(value, value_spatial_shapes, sampling_locations, attention_weights):
    return _run(value, sampling_locations, attention_weights)


# speedup vs baseline: 2.6344x; 1.1984x over previous
import jax
import jax.numpy as jnp
from jax import lax
from jax.experimental import pallas as pl
from jax.experimental.pallas import tpu as pltpu
from jax.experimental.pallas import tpu_sc as plsc

NC = 2
NS = 16
L = 16

H_SP = 64
W_SP = 64
HEADS = 8
D = 32
Q = 4096
P = 4
K = H_SP * W_SP
PAIRS = 32
NWORD = D // 2
R = 128
GROUPS = R // L
NCHUNK = Q // R


def _floor_i32(v):
    t = v.astype(jnp.int32)
    tf = t.astype(jnp.float32)
    return jnp.where(v < tf, t - 1, t)


def _body(slab_hbm, lc_hbm, aw_hbm, out_hbm, slab_v, loc_c, aw_c, out_v):
    wid = lax.axis_index("c") * NS + lax.axis_index("s")
    b = lax.shift_right_logical(wid, 3)
    h = lax.bitwise_and(wid, HEADS - 1)
    lanes = lax.iota(jnp.int32, L)

    pltpu.sync_copy(slab_hbm.at[wid], slab_v)

    def chunk_body(c, carry):
        pltpu.sync_copy(lc_hbm.at[b, h, 0, :, :, pl.ds(c * R, R)], loc_c)
        pltpu.sync_copy(aw_hbm.at[b, h, 0, :, pl.ds(c * R, R)], aw_c)

        def group_body(g, gcarry):
            wbfs = []
            tbases = []
            for p in range(P):
                xx = loc_c[p, 0, pl.ds(g * L, L)]
                yy = loc_c[p, 1, pl.ds(g * L, L)]
                aa = aw_c[p, pl.ds(g * L, L)]
                ix = xx * jnp.float32(W_SP) - 0.5
                iy = yy * jnp.float32(H_SP) - 0.5
                x0 = _floor_i32(ix)
                y0 = _floor_i32(iy)
                wx1 = ix - x0.astype(jnp.float32)
                wx0 = 1.0 - wx1
                wy1 = iy - y0.astype(jnp.float32)
                wy0 = 1.0 - wy1
                for sy in (0, 1):
                    yc = y0 + sy
                    wy = wy1 if sy else wy0
                    vy = (yc >= 0) & (yc <= H_SP - 1)
                    ycc = jnp.clip(yc, 0, H_SP - 1)
                    for sx in (0, 1):
                        xc = x0 + sx
                        wx = wx1 if sx else wx0
                        ok = vy & (xc >= 0) & (xc <= W_SP - 1)
                        xcc = jnp.clip(xc, 0, W_SP - 1)
                        t = ycc * W_SP + xcc
                        w = jnp.where(ok, aa * wx * wy, 0.0)
                        wbfs.append(
                            plsc.pack(w, w, format=plsc.PackFormat.INTERLEAVED)
                        )
                        tbases.append(t)
            for wp in range(NWORD):
                acc = jnp.zeros((2 * L,), jnp.bfloat16)
                accb = jnp.zeros((2 * L,), jnp.bfloat16)
                wpv = jnp.full((L,), wp, jnp.int32)
                for s in range(0, 16, 2):
                    worda = plsc.load_gather(slab_v, [wpv, tbases[s]])
                    wordb = plsc.load_gather(slab_v, [wpv, tbases[s + 1]])
                    acc = acc + plsc.bitcast(worda, jnp.bfloat16) * wbfs[s]
                    accb = accb + plsc.bitcast(wordb, jnp.bfloat16) * wbfs[s + 1]
                out_v[wp, pl.ds(g * L, L)] = plsc.bitcast(acc + accb, jnp.int32)
            return gcarry

        lax.fori_loop(0, GROUPS, group_body, 0)
        pltpu.sync_copy(out_v, out_hbm.at[wid, :, pl.ds(c * R, R)])
        return carry

    lax.fori_loop(0, NCHUNK, chunk_body, 0)


def _bf16_bits(x):
    u = lax.bitcast_convert_type(x, jnp.uint32)
    rounded = u + jnp.uint32(0x7FFF) + (lax.shift_right_logical(u, jnp.uint32(16)) & jnp.uint32(1))
    return lax.shift_right_logical(rounded, jnp.uint32(16)).astype(jnp.int32)


@jax.jit
def _run(value, loc6, aw5):
    vt = value.transpose(0, 2, 3, 1)
    lc = loc6.transpose(0, 2, 3, 4, 5, 1)
    aw = aw5.transpose(0, 2, 3, 4, 1)
    ve = _bf16_bits(vt[:, :, 0::2, :])
    vo = _bf16_bits(vt[:, :, 1::2, :])
    slab = (ve | lax.shift_left(vo, 16)).reshape(PAIRS, NWORD, Q)

    kfn = pl.kernel(
        _body,
        out_type=jax.ShapeDtypeStruct((PAIRS, NWORD, Q), jnp.int32),
        mesh=plsc.VectorSubcoreMesh(
            core_axis_name="c", subcore_axis_name="s",
            num_cores=NC, num_subcores=NS,
        ),
        scratch_types=[
            pltpu.VMEM((NWORD, Q), jnp.int32),
            pltpu.VMEM((P, 2, R), jnp.float32),
            pltpu.VMEM((P, R), jnp.float32),
            pltpu.VMEM((NWORD, R), jnp.int32),
        ],
        compiler_params=pltpu.CompilerParams(needs_layout_passes=False),
    )
    out = kfn(slab, lc, aw)
    out_bf = lax.bitcast_convert_type(
        out.reshape(4, HEADS, NWORD, Q), jnp.bfloat16)
    return (
        out_bf.astype(jnp.float32)
        .transpose(0, 3, 1, 2, 4)
        .reshape(4, Q, HEADS * D)
    )


def kernel(value, value_spatial_shapes, sampling_locations, attention_weights):
    return _run(value, sampling_locations, attention_weights)


# speedup vs baseline: 2.7785x; 1.0547x over previous
import jax
import jax.numpy as jnp
from jax import lax
from jax.experimental import pallas as pl
from jax.experimental.pallas import tpu as pltpu
from jax.experimental.pallas import tpu_sc as plsc

NC = 2
NS = 16
L = 16

H_SP = 64
W_SP = 64
HEADS = 8
D = 32
Q = 4096
P = 4
K = H_SP * W_SP
PAIRS = 32
NWORD = D // 2
R = 128
GROUPS = R // L
NCHUNK = Q // R


def _floor_i32(v):
    t = v.astype(jnp.int32)
    tf = t.astype(jnp.float32)
    return jnp.where(v < tf, t - 1, t)


def _body(slab_hbm, lc_hbm, aw_hbm, out_hbm, slab_v, loc_c, aw_c, out_v):
    wid = lax.axis_index("c") * NS + lax.axis_index("s")
    b = lax.shift_right_logical(wid, 3)
    h = lax.bitwise_and(wid, HEADS - 1)
    lanes = lax.iota(jnp.int32, L)

    pltpu.sync_copy(slab_hbm.at[wid], slab_v)

    def chunk_body(c, carry):
        pltpu.sync_copy(lc_hbm.at[b, h, 0, :, :, pl.ds(c * R, R)], loc_c)
        pltpu.sync_copy(aw_hbm.at[b, h, 0, :, pl.ds(c * R, R)], aw_c)

        def group_body(g, gcarry):
            wbfs = []
            tbases = []
            for p in range(P):
                xx = loc_c[p, 0, pl.ds(g * L, L)]
                yy = loc_c[p, 1, pl.ds(g * L, L)]
                aa = aw_c[p, pl.ds(g * L, L)]
                ix = xx * jnp.float32(W_SP) - 0.5
                iy = yy * jnp.float32(H_SP) - 0.5
                x0 = _floor_i32(ix)
                y0 = _floor_i32(iy)
                wx1 = ix - x0.astype(jnp.float32)
                wx0 = 1.0 - wx1
                wy1 = iy - y0.astype(jnp.float32)
                wy0 = 1.0 - wy1
                for sy in (0, 1):
                    yc = y0 + sy
                    wy = wy1 if sy else wy0
                    vy = (yc >= 0) & (yc <= H_SP - 1)
                    ycc = jnp.clip(yc, 0, H_SP - 1)
                    for sx in (0, 1):
                        xc = x0 + sx
                        wx = wx1 if sx else wx0
                        ok = vy & (xc >= 0) & (xc <= W_SP - 1)
                        xcc = jnp.clip(xc, 0, W_SP - 1)
                        t = ycc * W_SP + xcc
                        w = jnp.where(ok, aa * wx * wy, 0.0)
                        wbfs.append(
                            plsc.pack(w, w, format=plsc.PackFormat.INTERLEAVED)
                        )
                        tbases.append(t)
            for wp in range(NWORD):
                wpv = jnp.full((L,), wp, jnp.int32)
                words = [plsc.load_gather(slab_v, [wpv, tbases[s]])
                         for s in range(16)]
                accs = [jnp.zeros((2 * L,), jnp.bfloat16) for _ in range(4)]
                for s in range(16):
                    accs[s % 4] = accs[s % 4] + (
                        plsc.bitcast(words[s], jnp.bfloat16) * wbfs[s])
                out_v[wp, pl.ds(g * L, L)] = plsc.bitcast(
                    (accs[0] + accs[1]) + (accs[2] + accs[3]), jnp.int32)
            return gcarry

        lax.fori_loop(0, GROUPS, group_body, 0)
        pltpu.sync_copy(out_v, out_hbm.at[wid, :, pl.ds(c * R, R)])
        return carry

    lax.fori_loop(0, NCHUNK, chunk_body, 0)


def _bf16_bits(x):
    u = lax.bitcast_convert_type(x, jnp.uint32)
    rounded = u + jnp.uint32(0x7FFF) + (lax.shift_right_logical(u, jnp.uint32(16)) & jnp.uint32(1))
    return lax.shift_right_logical(rounded, jnp.uint32(16)).astype(jnp.int32)


@jax.jit
def _run(value, loc6, aw5):
    vt = value.transpose(0, 2, 3, 1)
    lc = loc6.transpose(0, 2, 3, 4, 5, 1)
    aw = aw5.transpose(0, 2, 3, 4, 1)
    bits = _bf16_bits(vt).reshape(4, HEADS, NWORD, 2, Q).astype(jnp.uint32)
    wsel = jnp.array([1, 65536], jnp.uint32).reshape(1, 1, 1, 2, 1)
    slab = lax.bitcast_convert_type(
        (bits * wsel).sum(axis=3, dtype=jnp.uint32), jnp.int32
    ).reshape(PAIRS, NWORD, Q)

    kfn = pl.kernel(
        _body,
        out_type=jax.ShapeDtypeStruct((PAIRS, NWORD, Q), jnp.int32),
        mesh=plsc.VectorSubcoreMesh(
            core_axis_name="c", subcore_axis_name="s",
            num_cores=NC, num_subcores=NS,
        ),
        scratch_types=[
            pltpu.VMEM((NWORD, Q), jnp.int32),
            pltpu.VMEM((P, 2, R), jnp.float32),
            pltpu.VMEM((P, R), jnp.float32),
            pltpu.VMEM((NWORD, R), jnp.int32),
        ],
        compiler_params=pltpu.CompilerParams(needs_layout_passes=False),
    )
    out = kfn(slab, lc, aw)
    out_bf = lax.bitcast_convert_type(
        out.reshape(4, HEADS, NWORD, Q), jnp.bfloat16)
    return (
        out_bf.astype(jnp.float32)
        .transpose(0, 3, 1, 2, 4)
        .reshape(4, Q, HEADS * D)
    )


def kernel(value, value_spatial_shapes, sampling_locations, attention_weights):
    return _run(value, sampling_locations, attention_weights)


# speedup vs baseline: 3.0868x; 1.1110x over previous
import jax
import jax.numpy as jnp
from jax import lax
from jax.experimental import pallas as pl
from jax.experimental.pallas import tpu as pltpu
from jax.experimental.pallas import tpu_sc as plsc

NC = 2
NS = 16
L = 16

H_SP = 64
W_SP = 64
HEADS = 8
D = 32
Q = 4096
P = 4
K = H_SP * W_SP
PAIRS = 32
NWORD = D // 2
R = 128
GROUPS = R // L
NCHUNK = Q // R


def _floor_i32(v):
    t = v.astype(jnp.int32)
    tf = t.astype(jnp.float32)
    return jnp.where(v < tf, t - 1, t)


def _body(slab_hbm, lc_hbm, aw_hbm, out_hbm, slab_v, loc_c, aw_c, out_v):
    wid = lax.axis_index("c") * NS + lax.axis_index("s")
    b = lax.shift_right_logical(wid, 3)
    h = lax.bitwise_and(wid, HEADS - 1)
    lanes = lax.iota(jnp.int32, L)

    pltpu.sync_copy(slab_hbm.at[wid], slab_v)

    def chunk_body(c, carry):
        pltpu.sync_copy(lc_hbm.at[b, h, 0, :, :, pl.ds(c * R, R)], loc_c)
        pltpu.sync_copy(aw_hbm.at[b, h, 0, :, pl.ds(c * R, R)], aw_c)

        def group_body(g, gcarry):
            wbfs = []
            tbases = []
            for p in range(P):
                xx = loc_c[p, 0, pl.ds(g * L, L)]
                yy = loc_c[p, 1, pl.ds(g * L, L)]
                aa = aw_c[p, pl.ds(g * L, L)]
                ix = xx * jnp.float32(W_SP) - 0.5
                iy = yy * jnp.float32(H_SP) - 0.5
                x0 = _floor_i32(ix)
                y0 = _floor_i32(iy)
                wx1 = ix - x0.astype(jnp.float32)
                wx0 = 1.0 - wx1
                wy1 = iy - y0.astype(jnp.float32)
                wy0 = 1.0 - wy1
                for sy in (0, 1):
                    yc = y0 + sy
                    wy = wy1 if sy else wy0
                    vy = (yc >= 0) & (yc <= H_SP - 1)
                    ycc = jnp.clip(yc, 0, H_SP - 1)
                    for sx in (0, 1):
                        xc = x0 + sx
                        wx = wx1 if sx else wx0
                        ok = vy & (xc >= 0) & (xc <= W_SP - 1)
                        xcc = jnp.clip(xc, 0, W_SP - 1)
                        t = ycc * W_SP + xcc
                        w = jnp.where(ok, aa * wx * wy, 0.0)
                        wbfs.append(
                            plsc.pack(w, w, format=plsc.PackFormat.INTERLEAVED)
                        )
                        tbases.append(t)
            for wp in range(NWORD):
                wpv = jnp.full((L,), wp, jnp.int32)
                words = [plsc.load_gather(slab_v, [wpv, tbases[s]])
                         for s in range(16)]
                accs = [jnp.zeros((2 * L,), jnp.bfloat16) for _ in range(4)]
                for s in range(16):
                    accs[s % 4] = accs[s % 4] + (
                        plsc.bitcast(words[s], jnp.bfloat16) * wbfs[s])
                out_v[wp, pl.ds(g * L, L)] = plsc.bitcast(
                    (accs[0] + accs[1]) + (accs[2] + accs[3]), jnp.int32)
            return gcarry

        lax.fori_loop(0, GROUPS, group_body, 0)
        pltpu.sync_copy(out_v, out_hbm.at[wid, :, pl.ds(c * R, R)])
        return carry

    lax.fori_loop(0, NCHUNK, chunk_body, 0)


def _bf16_bits(x):
    u = lax.bitcast_convert_type(x, jnp.uint32)
    rounded = u + jnp.uint32(0x7FFF) + (lax.shift_right_logical(u, jnp.uint32(16)) & jnp.uint32(1))
    return lax.shift_right_logical(rounded, jnp.uint32(16)).astype(jnp.int32)


@jax.jit
def _run(value, loc6, aw5):
    vt = value.transpose(0, 2, 3, 1)
    lc = loc6.transpose(0, 2, 3, 4, 5, 1)
    aw = aw5.transpose(0, 2, 3, 4, 1)
    bits = _bf16_bits(vt).reshape(4, HEADS, NWORD, 2 * Q)
    slab = (bits[..., :Q] | lax.shift_left(bits[..., Q:], 16)).reshape(
        PAIRS, NWORD, Q)

    kfn = pl.kernel(
        _body,
        out_type=jax.ShapeDtypeStruct((PAIRS, NWORD, Q), jnp.int32),
        mesh=plsc.VectorSubcoreMesh(
            core_axis_name="c", subcore_axis_name="s",
            num_cores=NC, num_subcores=NS,
        ),
        scratch_types=[
            pltpu.VMEM((NWORD, Q), jnp.int32),
            pltpu.VMEM((P, 2, R), jnp.float32),
            pltpu.VMEM((P, R), jnp.float32),
            pltpu.VMEM((NWORD, R), jnp.int32),
        ],
        compiler_params=pltpu.CompilerParams(needs_layout_passes=False),
    )
    out = kfn(slab, lc, aw)
    out_bf = lax.bitcast_convert_type(
        out.reshape(4, HEADS, NWORD, Q), jnp.bfloat16)
    return (
        out_bf.transpose(0, 3, 1, 2, 4)
        .reshape(4, Q, HEADS * D)
        .astype(jnp.float32)
    )


def kernel(value, value_spatial_shapes, sampling_locations, attention_weights):
    return _run(value, sampling_locations, attention_weights)
